# Initial kernel scaffold; baseline (speedup 1.0000x reference)
import jax
import jax.numpy as jnp
from jax.experimental import pallas as pl


def kernel(x, conv_w, conv_b):
    raise NotImplementedError("write your pallas kernel here")


# speedup vs baseline: 1.4238x; 1.4238x over previous
import jax
import jax.numpy as jnp
from jax.experimental import pallas as pl
from jax.experimental.pallas import tpu as pltpu

_N0 = 8
_NM = 7
_NS = 16
_C = 128
_PIX = 256
_HI = jax.lax.Precision.HIGHEST


def _conv_band_matrix(conv_w):
    idx = jnp.arange(16)
    dy = idx[:, None] - idx[None, :]
    oky = jnp.abs(dy) <= 3
    ky = jnp.clip(dy + 3, 0, 6)
    mats = []
    for d in range(2):
        wd = conv_w[0, d]
        m4 = wd[ky[:, None, :, None], ky[None, :, None, :]]
        m4 = jnp.where(oky[:, None, :, None] & oky[None, :, None, :], m4, 0.0)
        mats.append(m4.reshape(256, 256))
    return jnp.concatenate(mats, axis=0)


def _agg_kernel(x_ref, m_ref, b_ref, out_ref, t_ref):
    t_ref[0:_N0] = x_ref[0]
    bias = b_ref[0, 0]
    mband = m_ref[...]

    row16 = jax.lax.broadcasted_iota(jnp.int32, (_NS, _NS), 0)
    col16 = jax.lax.broadcasted_iota(jnp.int32, (_NS, _NS), 1)
    flat16 = row16 * _NS + col16
    inf = jnp.float32(jnp.inf)
    bigi = jnp.int32(2**30)

    ciota = jax.lax.broadcasted_iota(jnp.int32, (1, _NS), 1)

    t8 = x_ref[0]
    sq8 = jnp.sum(t8 * t8, axis=(1, 2))
    sq = jnp.concatenate(
        [sq8[None, :], jnp.zeros((1, _NS - _N0), jnp.float32)], axis=1)
    rows = []
    for m in range(_N0):
        g = jnp.sum(t8 * t8[m][None], axis=(1, 2))
        sqm = jnp.sum(t8[m] * t8[m])
        r = sq8 + sqm - 2.0 * g
        rows.append(jnp.concatenate(
            [r, jnp.full((_NS - _N0,), inf)])[None, :])
    rows += [jnp.full((1, _NS), inf)] * (_NS - _N0)
    D = jnp.concatenate(rows, axis=0)

    act_r = (row16 < _N0).astype(jnp.float32)
    act_c = (col16 < _N0).astype(jnp.float32)

    v = None
    for k in range(_NM):
        p = _N0 + k
        valid = (act_r > 0.5) & (act_c > 0.5) & (row16 > col16)
        deff = jnp.where(valid, D, inf)
        dmin = jnp.min(deff)
        fidx = jnp.min(jnp.where(deff == dmin, flat16, bigi))
        j = fidx // _NS
        i = fidx - j * _NS

        xl = t_ref[i]
        xr = t_ref[j]
        pair = jnp.concatenate([xl, xr], axis=1)
        v = jax.lax.dot_general(pair, mband, (((1,), (0,)), ((), ())),
                                precision=_HI) + bias
        t_ref[p] = v

        sq_p = jnp.sum(v * v)
        g = jnp.sum(t_ref[...] * v[None], axis=(1, 2))
        dnew = sq + sq_p - 2.0 * g[None, :]
        D = jnp.where(row16 == p, dnew, D)
        sq = jnp.where(ciota == p, sq_p, sq)

        act_r = jnp.where((row16 == i) | (row16 == j), 0.0, act_r)
        act_c = jnp.where((col16 == i) | (col16 == j), 0.0, act_c)
        act_r = jnp.where(row16 == p, 1.0, act_r)
        act_c = jnp.where(col16 == p, 1.0, act_c)

    out_ref[0] = v


def kernel(x, conv_w, conv_b):
    b, n0, c, w, h = x.shape
    xr = x.reshape(b, n0, c, w * h)
    mband = _conv_band_matrix(conv_w)
    bias = conv_b.reshape(1, 1)
    out = pl.pallas_call(
        _agg_kernel,
        grid=(b,),
        in_specs=[
            pl.BlockSpec((1, n0, c, w * h), lambda i: (i, 0, 0, 0)),
            pl.BlockSpec((2 * w * h, w * h), lambda i: (0, 0)),
            pl.BlockSpec((1, 1), lambda i: (0, 0)),
        ],
        out_specs=pl.BlockSpec((1, c, w * h), lambda i: (i, 0, 0)),
        out_shape=jax.ShapeDtypeStruct((b, c, w * h), jnp.float32),
        scratch_shapes=[pltpu.VMEM((_NS, c, w * h), jnp.float32)],
        compiler_params=pltpu.CompilerParams(
            dimension_semantics=("arbitrary",)),
    )(xr, mband, bias)
    return out.reshape(b, c, w, h)


# speedup vs baseline: 12.6725x; 8.9002x over previous
import jax
import jax.numpy as jnp
from jax.experimental import pallas as pl
from jax.experimental.pallas import tpu as pltpu

_N0 = 8
_NM = 7
_NS = 16
_C = 128
_PIX = 256
_HI = jax.lax.Precision.HIGHEST


def _conv_band_matrix(conv_w):
    eyes = jnp.stack([jnp.eye(16, k=3 - k, dtype=jnp.float32)
                      for k in range(7)])
    mats = []
    for d in range(2):
        m4 = jnp.einsum('kab,kl,lcd->acbd', eyes, conv_w[0, d], eyes,
                        precision=jax.lax.Precision.HIGHEST)
        mats.append(m4.reshape(256, 256))
    return jnp.concatenate(mats, axis=0)


def _agg_kernel(x_ref, m_ref, b_ref, out_ref, t_ref):
    t_ref[0:_N0] = x_ref[0]
    bias = b_ref[0, 0]
    mband = m_ref[...]

    row16 = jax.lax.broadcasted_iota(jnp.int32, (_NS, _NS), 0)
    col16 = jax.lax.broadcasted_iota(jnp.int32, (_NS, _NS), 1)
    flat16 = row16 * _NS + col16
    inf = jnp.float32(jnp.inf)
    bigi = jnp.int32(2**30)

    ciota = jax.lax.broadcasted_iota(jnp.int32, (1, _NS), 1)

    t8 = x_ref[0]
    sq8 = jnp.sum(t8 * t8, axis=(1, 2))
    sq = jnp.concatenate(
        [sq8[None, :], jnp.zeros((1, _NS - _N0), jnp.float32)], axis=1)
    rows = []
    for m in range(_N0):
        g = jnp.sum(t8 * t8[m][None], axis=(1, 2))
        sqm = jnp.sum(t8[m] * t8[m])
        r = sq8 + sqm - 2.0 * g
        rows.append(jnp.concatenate(
            [r, jnp.full((_NS - _N0,), inf)])[None, :])
    rows += [jnp.full((1, _NS), inf)] * (_NS - _N0)
    D = jnp.concatenate(rows, axis=0)

    act_r = (row16 < _N0).astype(jnp.float32)
    act_c = (col16 < _N0).astype(jnp.float32)

    v = None
    for k in range(_NM):
        p = _N0 + k
        valid = (act_r > 0.5) & (act_c > 0.5) & (row16 > col16)
        deff = jnp.where(valid, D, inf)
        dmin = jnp.min(deff)
        fidx = jnp.min(jnp.where(deff == dmin, flat16, bigi))
        j = fidx // _NS
        i = fidx - j * _NS

        xl = t_ref[i]
        xr = t_ref[j]
        pair = jnp.concatenate([xl, xr], axis=1)
        v = jax.lax.dot_general(pair, mband, (((1,), (0,)), ((), ())),
                                precision=_HI) + bias
        t_ref[p] = v

        sq_p = jnp.sum(v * v)
        g = jnp.sum(t_ref[...] * v[None], axis=(1, 2))
        dnew = sq + sq_p - 2.0 * g[None, :]
        D = jnp.where(row16 == p, dnew, D)
        sq = jnp.where(ciota == p, sq_p, sq)

        act_r = jnp.where((row16 == i) | (row16 == j), 0.0, act_r)
        act_c = jnp.where((col16 == i) | (col16 == j), 0.0, act_c)
        act_r = jnp.where(row16 == p, 1.0, act_r)
        act_c = jnp.where(col16 == p, 1.0, act_c)

    out_ref[0] = v


def kernel(x, conv_w, conv_b):
    b, n0, c, w, h = x.shape
    xr = x.reshape(b, n0, c, w * h)
    mband = _conv_band_matrix(conv_w)
    bias = conv_b.reshape(1, 1)
    out = pl.pallas_call(
        _agg_kernel,
        grid=(b,),
        in_specs=[
            pl.BlockSpec((1, n0, c, w * h), lambda i: (i, 0, 0, 0)),
            pl.BlockSpec((2 * w * h, w * h), lambda i: (0, 0)),
            pl.BlockSpec((1, 1), lambda i: (0, 0)),
        ],
        out_specs=pl.BlockSpec((1, c, w * h), lambda i: (i, 0, 0)),
        out_shape=jax.ShapeDtypeStruct((b, c, w * h), jnp.float32),
        scratch_shapes=[pltpu.VMEM((_NS, c, w * h), jnp.float32)],
        compiler_params=pltpu.CompilerParams(
            dimension_semantics=("arbitrary",)),
    )(xr, mband, bias)
    return out.reshape(b, c, w, h)


# speedup vs baseline: 16.4327x; 1.2967x over previous
import jax
import jax.numpy as jnp
from jax.experimental import pallas as pl
from jax.experimental.pallas import tpu as pltpu

_N0 = 8
_NM = 7
_NS = 16
_C = 128
_PIX = 256
_HI = jax.lax.Precision.HIGHEST


def _conv_band_matrix(conv_w):
    eyes = jnp.stack([jnp.eye(16, k=3 - k, dtype=jnp.float32)
                      for k in range(7)])
    mats = []
    for d in range(2):
        m4 = jnp.einsum('kab,kl,lcd->acbd', eyes, conv_w[0, d], eyes,
                        precision=jax.lax.Precision.HIGHEST)
        mats.append(m4.reshape(256, 256))
    return jnp.concatenate(mats, axis=0)


def _agg_kernel(x_ref, m_ref, b_ref, out_ref, t_ref):
    t_ref[0:_N0] = x_ref[0]
    bias = b_ref[0, 0]
    mband = m_ref[...]

    row16 = jax.lax.broadcasted_iota(jnp.int32, (_NS, _NS), 0)
    col16 = jax.lax.broadcasted_iota(jnp.int32, (_NS, _NS), 1)
    flat16 = row16 * _NS + col16
    inf = jnp.float32(jnp.inf)
    bigi = jnp.int32(2**30)

    ciota = jax.lax.broadcasted_iota(jnp.int32, (1, _NS), 1)

    t8 = x_ref[0]
    sq8 = jnp.sum(t8 * t8, axis=(1, 2))
    sq = jnp.concatenate(
        [sq8[None, :], jnp.zeros((1, _NS - _N0), jnp.float32)], axis=1)
    rows = []
    for m in range(_N0):
        g = jnp.sum(t8 * t8[m][None], axis=(1, 2))
        sqm = jnp.sum(t8[m] * t8[m])
        r = sq8 + sqm - 2.0 * g
        rows.append(jnp.concatenate(
            [r, jnp.full((_NS - _N0,), inf)])[None, :])
    rows += [jnp.full((1, _NS), inf)] * (_NS - _N0)
    D = jnp.concatenate(rows, axis=0)

    act_r = (row16 < _N0).astype(jnp.float32)
    act_c = (col16 < _N0).astype(jnp.float32)

    v = None
    for k in range(_NM):
        p = _N0 + k
        valid = (act_r > 0.5) & (act_c > 0.5) & (row16 > col16)
        deff = jnp.where(valid, D, inf)
        dmin = jnp.min(deff)
        fidx = jnp.min(jnp.where(deff == dmin, flat16, bigi))
        j = fidx // _NS
        i = fidx - j * _NS

        xl = t_ref[i]
        xr = t_ref[j]
        pair = jnp.concatenate([xl, xr], axis=1)
        v = jax.lax.dot_general(pair, mband, (((1,), (0,)), ((), ())),
                                precision=jax.lax.Precision.DEFAULT) + bias
        t_ref[p] = v

        sq_p = jnp.sum(v * v)
        g = jnp.sum(t_ref[0:p] * v[None], axis=(1, 2))
        g = jnp.concatenate([g, jnp.zeros((_NS - p,), jnp.float32)])
        dnew = sq + sq_p - 2.0 * g[None, :]
        D = jnp.where(row16 == p, dnew, D)
        sq = jnp.where(ciota == p, sq_p, sq)

        act_r = jnp.where((row16 == i) | (row16 == j), 0.0, act_r)
        act_c = jnp.where((col16 == i) | (col16 == j), 0.0, act_c)
        act_r = jnp.where(row16 == p, 1.0, act_r)
        act_c = jnp.where(col16 == p, 1.0, act_c)

    out_ref[0] = v


def kernel(x, conv_w, conv_b):
    b, n0, c, w, h = x.shape
    xr = x.reshape(b, n0, c, w * h)
    mband = _conv_band_matrix(conv_w)
    bias = conv_b.reshape(1, 1)
    out = pl.pallas_call(
        _agg_kernel,
        grid=(b,),
        in_specs=[
            pl.BlockSpec((1, n0, c, w * h), lambda i: (i, 0, 0, 0)),
            pl.BlockSpec((2 * w * h, w * h), lambda i: (0, 0)),
            pl.BlockSpec((1, 1), lambda i: (0, 0)),
        ],
        out_specs=pl.BlockSpec((1, c, w * h), lambda i: (i, 0, 0)),
        out_shape=jax.ShapeDtypeStruct((b, c, w * h), jnp.float32),
        scratch_shapes=[pltpu.VMEM((_NS, c, w * h), jnp.float32)],
        compiler_params=pltpu.CompilerParams(
            dimension_semantics=("arbitrary",)),
    )(xr, mband, bias)
    return out.reshape(b, c, w, h)
